# Initial kernel scaffold; baseline (speedup 1.0000x reference)
import jax
import jax.numpy as jnp
from jax.experimental import pallas as pl


def kernel(x, edge_index, W1, b1, W2, b2, fW1, fb1, fW2, fb2):
    raise NotImplementedError("write your pallas kernel here")


# speedup vs baseline: 3.4058x; 3.4058x over previous
import functools

import jax
import jax.numpy as jnp
from jax import lax
from jax.experimental import pallas as pl
from jax.experimental.pallas import tpu as pltpu
from jax.experimental.pallas import tpu_sc as plsc

N = 10000
NP = 10240
KNN = 20
ROWS = 256
NB = NP // ROWS

_PREC = jax.lax.Precision.DEFAULT


CW = 1280
NCH = NP // CW


def _extract(d, ids, nsel):
  vals, idxs = [], []
  for _ in range(nsel):
    m = jnp.min(d, axis=1, keepdims=True)
    am = jnp.min(jnp.where(d <= m, ids, jnp.int32(2 ** 30)),
                 axis=1, keepdims=True)
    vals.append(m)
    idxs.append(am)
    d = jnp.where(ids == am, jnp.inf, d)
  return jnp.concatenate(vals, axis=1), jnp.concatenate(idxs, axis=1)


def _topk_body(xb_ref, xtc_ref, idx_ref, vals_ref):
  i = pl.program_id(0)
  c = pl.program_id(1)
  xb = xb_ref[...]
  xtc = xtc_ref[...]
  sq_i = jnp.sum(xb * xb, axis=1, keepdims=True)
  sq_j = jnp.sum(xtc * xtc, axis=0, keepdims=True)
  cross = lax.dot_general(xb, xtc, (((1,), (0,)), ((), ())),
                          precision=_PREC,
                          preferred_element_type=jnp.float32)
  d = sq_i + sq_j - 2.0 * cross
  col = c * CW + lax.broadcasted_iota(jnp.int32, (ROWS, CW), 1)
  row = i * ROWS + lax.broadcasted_iota(jnp.int32, (ROWS, CW), 0)
  d = jnp.where(jnp.logical_or(col == row, col >= N), jnp.inf, d)
  cvals, cidx = _extract(d, col, KNN)
  prev_vals = jnp.where(c == 0, jnp.inf, vals_ref[...])
  prev_idx = jnp.where(c == 0, 0, idx_ref[...])
  mvals, midx = _extract(jnp.concatenate([prev_vals, cvals], axis=1),
                         jnp.concatenate([prev_idx, cidx], axis=1), KNN)
  vals_ref[...] = mvals
  idx_ref[...] = midx


def _topk(xp):
  dp = xp.shape[1]
  xt = xp.T
  return pl.pallas_call(
      _topk_body,
      grid=(NB, NCH),
      in_specs=[
          pl.BlockSpec((ROWS, dp), lambda i, c: (i, 0)),
          pl.BlockSpec((dp, CW), lambda i, c: (0, c)),
      ],
      out_specs=pl.BlockSpec((ROWS, KNN), lambda i, c: (i, 0)),
      out_shape=jax.ShapeDtypeStruct((NP, KNN), jnp.int32),
      scratch_shapes=[pltpu.VMEM((ROWS, KNN), jnp.float32)],
  )(xp, xt)


def _sc_gather(table, idx_flat):
  B = idx_flat.shape[0]
  dp = table.shape[1]
  NW = 32
  b_per_w = B // NW
  ch = 640
  n_ch = b_per_w // ch
  mesh = plsc.VectorSubcoreMesh(core_axis_name="c", subcore_axis_name="s")

  @functools.partial(
      pl.kernel, mesh=mesh,
      compiler_params=pltpu.CompilerParams(use_tc_tiling_on_sc=False),
      out_type=jax.ShapeDtypeStruct((B, dp), jnp.float32),
      scratch_types=[
          pltpu.VMEM((ch,), jnp.int32),
          pltpu.VMEM((ch, dp), jnp.float32),
          pltpu.SemaphoreType.DMA,
      ],
  )
  def k(table_hbm, idx_hbm, out_hbm, idx_v, rows_v, sem):
    wid = lax.axis_index("s") * 2 + lax.axis_index("c")

    def body(t, carry):
      base = wid * b_per_w + t * ch
      pltpu.sync_copy(idx_hbm.at[pl.ds(base, ch)], idx_v)
      pltpu.async_copy(table_hbm.at[idx_v], rows_v, sem).wait()
      pltpu.sync_copy(rows_v, out_hbm.at[pl.ds(base, ch)])
      return carry

    lax.fori_loop(0, n_ch, body, 0)

  return k(table, idx_flat)


def _edge_mlp_body(xb_ref, xjt_ref, wa_ref, wb_ref, b_ref, out_ref):
  xb = xb_ref[...]
  base = lax.dot_general(xb, wa_ref[...], (((1,), (0,)), ((), ())),
                         precision=_PREC,
                         preferred_element_type=jnp.float32) + b_ref[...]
  wb = wb_ref[...]
  h = None
  for j in range(KNN):
    dj = xjt_ref[j] - xb
    hj = base + lax.dot_general(dj, wb, (((1,), (0,)), ((), ())),
                                precision=_PREC,
                                preferred_element_type=jnp.float32)
    hj = jnp.maximum(hj, 0.0)
    h = hj if h is None else jnp.maximum(h, hj)
  out_ref[...] = h


def _edge_mlp_final_body(xb_ref, xjt_ref, wa_ref, wb_ref, b_ref,
                         fw1_ref, fb1_ref, fw2_ref, fb2_ref, out_ref):
  xb = xb_ref[...]
  base = lax.dot_general(xb, wa_ref[...], (((1,), (0,)), ((), ())),
                         precision=_PREC,
                         preferred_element_type=jnp.float32) + b_ref[...]
  wb = wb_ref[...]
  h = None
  for j in range(KNN):
    dj = xjt_ref[j] - xb
    hj = base + lax.dot_general(dj, wb, (((1,), (0,)), ((), ())),
                                precision=_PREC,
                                preferred_element_type=jnp.float32)
    hj = jnp.maximum(hj, 0.0)
    h = hj if h is None else jnp.maximum(h, hj)
  t1 = lax.dot_general(h, fw1_ref[...], (((1,), (0,)), ((), ())),
                       precision=_PREC,
                       preferred_element_type=jnp.float32) + fb1_ref[...]
  t1 = jnp.maximum(t1, 0.0)
  out_ref[...] = lax.dot_general(t1, fw2_ref[...], (((1,), (0,)), ((), ())),
                                 precision=_PREC,
                                 preferred_element_type=jnp.float32) + fb2_ref[...]


def _edge_mlp(xp, xjt, wa, wb, b):
  dp = xp.shape[1]
  c = wa.shape[1]
  return pl.pallas_call(
      _edge_mlp_body,
      grid=(NB,),
      in_specs=[
          pl.BlockSpec((ROWS, dp), lambda i: (i, 0)),
          pl.BlockSpec((KNN, ROWS, dp), lambda i: (0, i, 0)),
          pl.BlockSpec((dp, c), lambda i: (0, 0)),
          pl.BlockSpec((dp, c), lambda i: (0, 0)),
          pl.BlockSpec((1, c), lambda i: (0, 0)),
      ],
      out_specs=pl.BlockSpec((ROWS, c), lambda i: (i, 0)),
      out_shape=jax.ShapeDtypeStruct((NP, c), jnp.float32),
  )(xp, xjt, wa, wb, b)


def _edge_mlp_final(xp, xjt, wa, wb, b, fw1, fb1, fw2, fb2):
  dp = xp.shape[1]
  c = wa.shape[1]
  co = fw2.shape[1]
  return pl.pallas_call(
      _edge_mlp_final_body,
      grid=(NB,),
      in_specs=[
          pl.BlockSpec((ROWS, dp), lambda i: (i, 0)),
          pl.BlockSpec((KNN, ROWS, dp), lambda i: (0, i, 0)),
          pl.BlockSpec((dp, c), lambda i: (0, 0)),
          pl.BlockSpec((dp, c), lambda i: (0, 0)),
          pl.BlockSpec((1, c), lambda i: (0, 0)),
          pl.BlockSpec((c, c), lambda i: (0, 0)),
          pl.BlockSpec((1, c), lambda i: (0, 0)),
          pl.BlockSpec((c, co), lambda i: (0, 0)),
          pl.BlockSpec((1, co), lambda i: (0, 0)),
      ],
      out_specs=pl.BlockSpec((ROWS, co), lambda i: (i, 0)),
      out_shape=jax.ShapeDtypeStruct((NP, co), jnp.float32),
  )(xp, xjt, wa, wb, b, fw1, fb1, fw2, fb2)


def kernel(x, edge_index, W1, b1, W2, b2, fW1, fb1, fW2, fb2):
  del edge_index

  dp1 = 16
  xp = jnp.zeros((NP, dp1), jnp.float32).at[:N, :3].set(x)
  idx1 = _topk(xp)
  idx1_flat = idx1.T.reshape(-1)
  xj1 = _sc_gather(xp, idx1_flat).reshape(KNN, NP, dp1)
  w1a = jnp.zeros((dp1, 64), jnp.float32).at[:3].set(W1[:3])
  w1b = jnp.zeros((dp1, 64), jnp.float32).at[:3].set(W1[3:])
  h1 = _edge_mlp(xp, xj1, w1a, w1b, b1.reshape(1, -1))

  idx2 = _topk(h1)
  idx2_flat = idx2.T.reshape(-1)
  xj2 = _sc_gather(h1, idx2_flat).reshape(KNN, NP, 64)
  out = _edge_mlp_final(h1, xj2, W2[:64], W2[64:], b2.reshape(1, -1),
                        fW1, fb1.reshape(1, -1), fW2, fb2.reshape(1, -1))
  return out[:N]


# speedup vs baseline: 6.0052x; 1.7632x over previous
import functools

import jax
import jax.numpy as jnp
from jax import lax
from jax.experimental import pallas as pl
from jax.experimental.pallas import tpu as pltpu
from jax.experimental.pallas import tpu_sc as plsc

N = 10000
NP = 10240
KNN = 20
ROWS = 256
NB = NP // ROWS

_PREC = jax.lax.Precision.DEFAULT


CW = 2560
NCH = NP // CW


def _extract(d, ids, nsel):
  vals, idxs = [], []
  for _ in range(nsel):
    m = jnp.min(d, axis=1, keepdims=True)
    am = jnp.min(jnp.where(d <= m, ids, jnp.float32(3e38)),
                 axis=1, keepdims=True)
    vals.append(m)
    idxs.append(am)
    d = jnp.where(ids == am, jnp.inf, d)
  return jnp.concatenate(vals, axis=1), jnp.concatenate(idxs, axis=1)


def _topk_body(xb_ref, xtc_ref, idx_ref, vals_ref, fidx_ref):
  i = pl.program_id(0)
  c = pl.program_id(1)
  xb = xb_ref[...]
  xtc = xtc_ref[...]
  sq_i = jnp.sum(xb * xb, axis=1, keepdims=True)
  sq_j = jnp.sum(xtc * xtc, axis=0, keepdims=True)
  cross = lax.dot_general(xb, xtc, (((1,), (0,)), ((), ())),
                          precision=_PREC,
                          preferred_element_type=jnp.float32)
  d = sq_i + sq_j - 2.0 * cross
  col = c * CW + lax.broadcasted_iota(jnp.int32, (ROWS, CW), 1)
  row = i * ROWS + lax.broadcasted_iota(jnp.int32, (ROWS, CW), 0)
  d = jnp.where(jnp.logical_or(col == row, col >= N), jnp.inf, d)
  colf = col.astype(jnp.float32)
  cvals, cidx = _extract(d, colf, KNN)
  prev_vals = jnp.where(c == 0, jnp.inf, vals_ref[...])
  prev_idx = jnp.where(c == 0, 0.0, fidx_ref[...])
  mvals, midx = _extract(jnp.concatenate([prev_vals, cvals], axis=1),
                         jnp.concatenate([prev_idx, cidx], axis=1), KNN)
  vals_ref[...] = mvals
  fidx_ref[...] = midx
  idx_ref[...] = midx.astype(jnp.int32)


def _topk(xp):
  dp = xp.shape[1]
  xt = xp.T
  return pl.pallas_call(
      _topk_body,
      grid=(NB, NCH),
      in_specs=[
          pl.BlockSpec((ROWS, dp), lambda i, c: (i, 0)),
          pl.BlockSpec((dp, CW), lambda i, c: (0, c)),
      ],
      out_specs=pl.BlockSpec((ROWS, KNN), lambda i, c: (i, 0)),
      out_shape=jax.ShapeDtypeStruct((NP, KNN), jnp.int32),
      scratch_shapes=[pltpu.VMEM((ROWS, KNN), jnp.float32),
                      pltpu.VMEM((ROWS, KNN), jnp.float32)],
  )(xp, xt)


def _sc_gather(table, idx_flat):
  B = idx_flat.shape[0]
  dp = table.shape[1]
  NW = 32
  b_per_w = B // NW
  ch = 640
  n_ch = b_per_w // ch
  mesh = plsc.VectorSubcoreMesh(core_axis_name="c", subcore_axis_name="s")

  @functools.partial(
      pl.kernel, mesh=mesh,
      compiler_params=pltpu.CompilerParams(use_tc_tiling_on_sc=False),
      out_type=jax.ShapeDtypeStruct((B, dp), jnp.float32),
      scratch_types=[
          pltpu.VMEM((ch,), jnp.int32),
          pltpu.VMEM((ch, dp), jnp.float32),
          pltpu.SemaphoreType.DMA,
      ],
  )
  def k(table_hbm, idx_hbm, out_hbm, idx_v, rows_v, sem):
    wid = lax.axis_index("s") * 2 + lax.axis_index("c")

    def body(t, carry):
      base = wid * b_per_w + t * ch
      pltpu.sync_copy(idx_hbm.at[pl.ds(base, ch)], idx_v)
      pltpu.async_copy(table_hbm.at[idx_v], rows_v, sem).wait()
      pltpu.sync_copy(rows_v, out_hbm.at[pl.ds(base, ch)])
      return carry

    lax.fori_loop(0, n_ch, body, 0)

  return k(table, idx_flat)


def _edge_mlp_body(xb_ref, xjt_ref, wa_ref, wb_ref, b_ref, out_ref):
  xb = xb_ref[...]
  base = lax.dot_general(xb, wa_ref[...], (((1,), (0,)), ((), ())),
                         precision=_PREC,
                         preferred_element_type=jnp.float32) + b_ref[...]
  wb = wb_ref[...]
  h = None
  for j in range(KNN):
    dj = xjt_ref[j] - xb
    hj = base + lax.dot_general(dj, wb, (((1,), (0,)), ((), ())),
                                precision=_PREC,
                                preferred_element_type=jnp.float32)
    hj = jnp.maximum(hj, 0.0)
    h = hj if h is None else jnp.maximum(h, hj)
  out_ref[...] = h


def _edge_mlp_final_body(xb_ref, xjt_ref, wa_ref, wb_ref, b_ref,
                         fw1_ref, fb1_ref, fw2_ref, fb2_ref, out_ref):
  xb = xb_ref[...]
  base = lax.dot_general(xb, wa_ref[...], (((1,), (0,)), ((), ())),
                         precision=_PREC,
                         preferred_element_type=jnp.float32) + b_ref[...]
  wb = wb_ref[...]
  h = None
  for j in range(KNN):
    dj = xjt_ref[j] - xb
    hj = base + lax.dot_general(dj, wb, (((1,), (0,)), ((), ())),
                                precision=_PREC,
                                preferred_element_type=jnp.float32)
    hj = jnp.maximum(hj, 0.0)
    h = hj if h is None else jnp.maximum(h, hj)
  t1 = lax.dot_general(h, fw1_ref[...], (((1,), (0,)), ((), ())),
                       precision=_PREC,
                       preferred_element_type=jnp.float32) + fb1_ref[...]
  t1 = jnp.maximum(t1, 0.0)
  out_ref[...] = lax.dot_general(t1, fw2_ref[...], (((1,), (0,)), ((), ())),
                                 precision=_PREC,
                                 preferred_element_type=jnp.float32) + fb2_ref[...]


def _edge_mlp(xp, xjt, wa, wb, b):
  dp = xp.shape[1]
  c = wa.shape[1]
  return pl.pallas_call(
      _edge_mlp_body,
      grid=(NB,),
      in_specs=[
          pl.BlockSpec((ROWS, dp), lambda i: (i, 0)),
          pl.BlockSpec((KNN, ROWS, dp), lambda i: (0, i, 0)),
          pl.BlockSpec((dp, c), lambda i: (0, 0)),
          pl.BlockSpec((dp, c), lambda i: (0, 0)),
          pl.BlockSpec((1, c), lambda i: (0, 0)),
      ],
      out_specs=pl.BlockSpec((ROWS, c), lambda i: (i, 0)),
      out_shape=jax.ShapeDtypeStruct((NP, c), jnp.float32),
  )(xp, xjt, wa, wb, b)


def _edge_mlp_final(xp, xjt, wa, wb, b, fw1, fb1, fw2, fb2):
  dp = xp.shape[1]
  c = wa.shape[1]
  co = fw2.shape[1]
  return pl.pallas_call(
      _edge_mlp_final_body,
      grid=(NB,),
      in_specs=[
          pl.BlockSpec((ROWS, dp), lambda i: (i, 0)),
          pl.BlockSpec((KNN, ROWS, dp), lambda i: (0, i, 0)),
          pl.BlockSpec((dp, c), lambda i: (0, 0)),
          pl.BlockSpec((dp, c), lambda i: (0, 0)),
          pl.BlockSpec((1, c), lambda i: (0, 0)),
          pl.BlockSpec((c, c), lambda i: (0, 0)),
          pl.BlockSpec((1, c), lambda i: (0, 0)),
          pl.BlockSpec((c, co), lambda i: (0, 0)),
          pl.BlockSpec((1, co), lambda i: (0, 0)),
      ],
      out_specs=pl.BlockSpec((ROWS, co), lambda i: (i, 0)),
      out_shape=jax.ShapeDtypeStruct((NP, co), jnp.float32),
  )(xp, xjt, wa, wb, b, fw1, fb1, fw2, fb2)


def kernel(x, edge_index, W1, b1, W2, b2, fW1, fb1, fW2, fb2):
  del edge_index

  dp1 = 16
  xp = jnp.zeros((NP, dp1), jnp.float32).at[:N, :3].set(x)
  idx1 = _topk(xp)
  idx1_flat = idx1.T.reshape(-1)
  xj1 = _sc_gather(xp, idx1_flat).reshape(KNN, NP, dp1)
  w1a = jnp.zeros((dp1, 64), jnp.float32).at[:3].set(W1[:3])
  w1b = jnp.zeros((dp1, 64), jnp.float32).at[:3].set(W1[3:])
  h1 = _edge_mlp(xp, xj1, w1a, w1b, b1.reshape(1, -1))

  idx2 = _topk(h1)
  idx2_flat = idx2.T.reshape(-1)
  xj2 = _sc_gather(h1, idx2_flat).reshape(KNN, NP, 64)
  out = _edge_mlp_final(h1, xj2, W2[:64], W2[64:], b2.reshape(1, -1),
                        fW1, fb1.reshape(1, -1), fW2, fb2.reshape(1, -1))
  return out[:N]


# speedup vs baseline: 7.1467x; 1.1901x over previous
import functools

import jax
import jax.numpy as jnp
from jax import lax
from jax.experimental import pallas as pl
from jax.experimental.pallas import tpu as pltpu
from jax.experimental.pallas import tpu_sc as plsc

N = 10000
NP = 10240
KNN = 20
ROWS = 256
NB = NP // ROWS

_PREC = jax.lax.Precision.DEFAULT


CW = 2560
NCH = NP // CW


def _extract(d, ids, nsel):
  vals, idxs = [], []
  for _ in range(nsel):
    m = jnp.min(d, axis=1, keepdims=True)
    am = jnp.min(jnp.where(d <= m, ids, jnp.float32(3e38)),
                 axis=1, keepdims=True)
    vals.append(m)
    idxs.append(am)
    d = jnp.where(ids == am, jnp.inf, d)
  return jnp.concatenate(vals, axis=1), jnp.concatenate(idxs, axis=1)


def _topk_chunk_body(xb_ref, xtc_ref, cv_ref, ci_ref):
  i = pl.program_id(0)
  c = pl.program_id(1)
  xb = xb_ref[...]
  xtc = xtc_ref[...]
  sq_i = jnp.sum(xb * xb, axis=1, keepdims=True)
  sq_j = jnp.sum(xtc * xtc, axis=0, keepdims=True)
  cross = lax.dot_general(xb, xtc, (((1,), (0,)), ((), ())),
                          precision=_PREC,
                          preferred_element_type=jnp.float32)
  d = sq_i + sq_j - 2.0 * cross
  col = c * CW + lax.broadcasted_iota(jnp.int32, (ROWS, CW), 1)
  row = i * ROWS + lax.broadcasted_iota(jnp.int32, (ROWS, CW), 0)
  d = jnp.where(jnp.logical_or(col == row, col >= N), jnp.inf, d)
  colf = col.astype(jnp.float32)
  cvals, cidx = _extract(d, colf, KNN)
  cv_ref[0] = cvals
  ci_ref[0] = cidx


def _topk_merge_body(cv_ref, ci_ref, idx_ref):
  cvs = cv_ref[...]
  cis = ci_ref[...]
  v = jnp.concatenate([cvs[k] for k in range(NCH)], axis=1)
  ids = jnp.concatenate([cis[k] for k in range(NCH)], axis=1)
  _, midx = _extract(v, ids, KNN)
  idx_ref[...] = midx.astype(jnp.int32)


def _topk(xp):
  dp = xp.shape[1]
  xt = xp.T
  cv, ci = pl.pallas_call(
      _topk_chunk_body,
      grid=(NB, NCH),
      in_specs=[
          pl.BlockSpec((ROWS, dp), lambda i, c: (i, 0)),
          pl.BlockSpec((dp, CW), lambda i, c: (0, c)),
      ],
      out_specs=[
          pl.BlockSpec((1, ROWS, KNN), lambda i, c: (c, i, 0)),
          pl.BlockSpec((1, ROWS, KNN), lambda i, c: (c, i, 0)),
      ],
      out_shape=[
          jax.ShapeDtypeStruct((NCH, NP, KNN), jnp.float32),
          jax.ShapeDtypeStruct((NCH, NP, KNN), jnp.float32),
      ],
  )(xp, xt)
  return pl.pallas_call(
      _topk_merge_body,
      grid=(NB,),
      in_specs=[
          pl.BlockSpec((NCH, ROWS, KNN), lambda i: (0, i, 0)),
          pl.BlockSpec((NCH, ROWS, KNN), lambda i: (0, i, 0)),
      ],
      out_specs=pl.BlockSpec((ROWS, KNN), lambda i: (i, 0)),
      out_shape=jax.ShapeDtypeStruct((NP, KNN), jnp.int32),
  )(cv, ci)


def _sc_gather(table, idx_flat):
  B = idx_flat.shape[0]
  dp = table.shape[1]
  NW = 32
  b_per_w = B // NW
  ch = 640
  n_ch = b_per_w // ch
  mesh = plsc.VectorSubcoreMesh(core_axis_name="c", subcore_axis_name="s")

  @functools.partial(
      pl.kernel, mesh=mesh,
      compiler_params=pltpu.CompilerParams(use_tc_tiling_on_sc=False),
      out_type=jax.ShapeDtypeStruct((B, dp), jnp.float32),
      scratch_types=[
          pltpu.VMEM((ch,), jnp.int32),
          pltpu.VMEM((ch, dp), jnp.float32),
          pltpu.SemaphoreType.DMA,
      ],
  )
  def k(table_hbm, idx_hbm, out_hbm, idx_v, rows_v, sem):
    wid = lax.axis_index("s") * 2 + lax.axis_index("c")

    def body(t, carry):
      base = wid * b_per_w + t * ch
      pltpu.sync_copy(idx_hbm.at[pl.ds(base, ch)], idx_v)
      pltpu.async_copy(table_hbm.at[idx_v], rows_v, sem).wait()
      pltpu.sync_copy(rows_v, out_hbm.at[pl.ds(base, ch)])
      return carry

    lax.fori_loop(0, n_ch, body, 0)

  return k(table, idx_flat)


def _edge_mlp_body(xb_ref, xjt_ref, wa_ref, wb_ref, b_ref, out_ref):
  xb = xb_ref[...]
  base = lax.dot_general(xb, wa_ref[...], (((1,), (0,)), ((), ())),
                         precision=_PREC,
                         preferred_element_type=jnp.float32) + b_ref[...]
  wb = wb_ref[...]
  h = None
  for j in range(KNN):
    dj = xjt_ref[j] - xb
    hj = base + lax.dot_general(dj, wb, (((1,), (0,)), ((), ())),
                                precision=_PREC,
                                preferred_element_type=jnp.float32)
    hj = jnp.maximum(hj, 0.0)
    h = hj if h is None else jnp.maximum(h, hj)
  out_ref[...] = h


def _edge_mlp_final_body(xb_ref, xjt_ref, wa_ref, wb_ref, b_ref,
                         fw1_ref, fb1_ref, fw2_ref, fb2_ref, out_ref):
  xb = xb_ref[...]
  base = lax.dot_general(xb, wa_ref[...], (((1,), (0,)), ((), ())),
                         precision=_PREC,
                         preferred_element_type=jnp.float32) + b_ref[...]
  wb = wb_ref[...]
  h = None
  for j in range(KNN):
    dj = xjt_ref[j] - xb
    hj = base + lax.dot_general(dj, wb, (((1,), (0,)), ((), ())),
                                precision=_PREC,
                                preferred_element_type=jnp.float32)
    hj = jnp.maximum(hj, 0.0)
    h = hj if h is None else jnp.maximum(h, hj)
  t1 = lax.dot_general(h, fw1_ref[...], (((1,), (0,)), ((), ())),
                       precision=_PREC,
                       preferred_element_type=jnp.float32) + fb1_ref[...]
  t1 = jnp.maximum(t1, 0.0)
  out_ref[...] = lax.dot_general(t1, fw2_ref[...], (((1,), (0,)), ((), ())),
                                 precision=_PREC,
                                 preferred_element_type=jnp.float32) + fb2_ref[...]


def _edge_mlp(xp, xjt, wa, wb, b):
  dp = xp.shape[1]
  c = wa.shape[1]
  return pl.pallas_call(
      _edge_mlp_body,
      grid=(NB,),
      in_specs=[
          pl.BlockSpec((ROWS, dp), lambda i: (i, 0)),
          pl.BlockSpec((KNN, ROWS, dp), lambda i: (0, i, 0)),
          pl.BlockSpec((dp, c), lambda i: (0, 0)),
          pl.BlockSpec((dp, c), lambda i: (0, 0)),
          pl.BlockSpec((1, c), lambda i: (0, 0)),
      ],
      out_specs=pl.BlockSpec((ROWS, c), lambda i: (i, 0)),
      out_shape=jax.ShapeDtypeStruct((NP, c), jnp.float32),
  )(xp, xjt, wa, wb, b)


def _edge_mlp_final(xp, xjt, wa, wb, b, fw1, fb1, fw2, fb2):
  dp = xp.shape[1]
  c = wa.shape[1]
  co = fw2.shape[1]
  return pl.pallas_call(
      _edge_mlp_final_body,
      grid=(NB,),
      in_specs=[
          pl.BlockSpec((ROWS, dp), lambda i: (i, 0)),
          pl.BlockSpec((KNN, ROWS, dp), lambda i: (0, i, 0)),
          pl.BlockSpec((dp, c), lambda i: (0, 0)),
          pl.BlockSpec((dp, c), lambda i: (0, 0)),
          pl.BlockSpec((1, c), lambda i: (0, 0)),
          pl.BlockSpec((c, c), lambda i: (0, 0)),
          pl.BlockSpec((1, c), lambda i: (0, 0)),
          pl.BlockSpec((c, co), lambda i: (0, 0)),
          pl.BlockSpec((1, co), lambda i: (0, 0)),
      ],
      out_specs=pl.BlockSpec((ROWS, co), lambda i: (i, 0)),
      out_shape=jax.ShapeDtypeStruct((NP, co), jnp.float32),
  )(xp, xjt, wa, wb, b, fw1, fb1, fw2, fb2)


def kernel(x, edge_index, W1, b1, W2, b2, fW1, fb1, fW2, fb2):
  del edge_index

  dp1 = 16
  xp = jnp.zeros((NP, dp1), jnp.float32).at[:N, :3].set(x)
  idx1 = _topk(xp)
  idx1_flat = idx1.T.reshape(-1)
  xj1 = _sc_gather(xp, idx1_flat).reshape(KNN, NP, dp1)
  w1a = jnp.zeros((dp1, 64), jnp.float32).at[:3].set(W1[:3])
  w1b = jnp.zeros((dp1, 64), jnp.float32).at[:3].set(W1[3:])
  h1 = _edge_mlp(xp, xj1, w1a, w1b, b1.reshape(1, -1))

  idx2 = _topk(h1)
  idx2_flat = idx2.T.reshape(-1)
  xj2 = _sc_gather(h1, idx2_flat).reshape(KNN, NP, 64)
  out = _edge_mlp_final(h1, xj2, W2[:64], W2[64:], b2.reshape(1, -1),
                        fW1, fb1.reshape(1, -1), fW2, fb2.reshape(1, -1))
  return out[:N]


# speedup vs baseline: 7.5267x; 1.0532x over previous
import functools

import jax
import jax.numpy as jnp
from jax import lax
from jax.experimental import pallas as pl
from jax.experimental.pallas import tpu as pltpu
from jax.experimental.pallas import tpu_sc as plsc

N = 10000
NP = 10240
KNN = 20
ROWS = 512
NB = NP // ROWS

_PREC = jax.lax.Precision.DEFAULT


CW = 2560
NCH = NP // CW


def _extract(d, ids, nsel):
  vals, idxs = [], []
  for _ in range(nsel):
    m = jnp.min(d, axis=1, keepdims=True)
    am = jnp.min(jnp.where(d <= m, ids, jnp.float32(3e38)),
                 axis=1, keepdims=True)
    vals.append(m)
    idxs.append(am)
    d = jnp.where(ids == am, jnp.inf, d)
  return jnp.concatenate(vals, axis=1), jnp.concatenate(idxs, axis=1)


def _topk_chunk_body(xb_ref, xtc_ref, cv_ref, ci_ref):
  i = pl.program_id(0)
  c = pl.program_id(1)
  xb = xb_ref[...]
  xtc = xtc_ref[...]
  sq_i = jnp.sum(xb * xb, axis=1, keepdims=True)
  sq_j = jnp.sum(xtc * xtc, axis=0, keepdims=True)
  cross = lax.dot_general(xb, xtc, (((1,), (0,)), ((), ())),
                          precision=_PREC,
                          preferred_element_type=jnp.float32)
  d = sq_i + sq_j - 2.0 * cross
  col = c * CW + lax.broadcasted_iota(jnp.int32, (ROWS, CW), 1)
  row = i * ROWS + lax.broadcasted_iota(jnp.int32, (ROWS, CW), 0)
  d = jnp.where(jnp.logical_or(col == row, col >= N), jnp.inf, d)
  colf = col.astype(jnp.float32)
  cvals, cidx = _extract(d, colf, KNN)
  cv_ref[0] = cvals
  ci_ref[0] = cidx


def _topk_merge_body(cv_ref, ci_ref, idx_ref):
  cvs = cv_ref[...]
  cis = ci_ref[...]
  v = jnp.concatenate([cvs[k] for k in range(NCH)], axis=1)
  ids = jnp.concatenate([cis[k] for k in range(NCH)], axis=1)
  _, midx = _extract(v, ids, KNN)
  idx_ref[...] = midx.astype(jnp.int32)


def _topk(xp):
  dp = xp.shape[1]
  xt = xp.T
  cv, ci = pl.pallas_call(
      _topk_chunk_body,
      grid=(NB, NCH),
      in_specs=[
          pl.BlockSpec((ROWS, dp), lambda i, c: (i, 0)),
          pl.BlockSpec((dp, CW), lambda i, c: (0, c)),
      ],
      out_specs=[
          pl.BlockSpec((1, ROWS, KNN), lambda i, c: (c, i, 0)),
          pl.BlockSpec((1, ROWS, KNN), lambda i, c: (c, i, 0)),
      ],
      out_shape=[
          jax.ShapeDtypeStruct((NCH, NP, KNN), jnp.float32),
          jax.ShapeDtypeStruct((NCH, NP, KNN), jnp.float32),
      ],
  )(xp, xt)
  return pl.pallas_call(
      _topk_merge_body,
      grid=(NB,),
      in_specs=[
          pl.BlockSpec((NCH, ROWS, KNN), lambda i: (0, i, 0)),
          pl.BlockSpec((NCH, ROWS, KNN), lambda i: (0, i, 0)),
      ],
      out_specs=pl.BlockSpec((ROWS, KNN), lambda i: (i, 0)),
      out_shape=jax.ShapeDtypeStruct((NP, KNN), jnp.int32),
  )(cv, ci)


def _sc_gather(table, idx_flat):
  B = idx_flat.shape[0]
  dp = table.shape[1]
  NW = 32
  b_per_w = B // NW
  ch = 640
  n_ch = b_per_w // ch
  mesh = plsc.VectorSubcoreMesh(core_axis_name="c", subcore_axis_name="s")

  @functools.partial(
      pl.kernel, mesh=mesh,
      compiler_params=pltpu.CompilerParams(use_tc_tiling_on_sc=False),
      out_type=jax.ShapeDtypeStruct((B, dp), jnp.float32),
      scratch_types=[
          pltpu.VMEM((ch,), jnp.int32),
          pltpu.VMEM((ch, dp), jnp.float32),
          pltpu.SemaphoreType.DMA,
      ],
  )
  def k(table_hbm, idx_hbm, out_hbm, idx_v, rows_v, sem):
    wid = lax.axis_index("s") * 2 + lax.axis_index("c")

    def body(t, carry):
      base = wid * b_per_w + t * ch
      pltpu.sync_copy(idx_hbm.at[pl.ds(base, ch)], idx_v)
      pltpu.async_copy(table_hbm.at[idx_v], rows_v, sem).wait()
      pltpu.sync_copy(rows_v, out_hbm.at[pl.ds(base, ch)])
      return carry

    lax.fori_loop(0, n_ch, body, 0)

  return k(table, idx_flat)


def _edge_mlp_body(xb_ref, xjt_ref, wa_ref, wb_ref, b_ref, out_ref):
  xb = xb_ref[...]
  base = lax.dot_general(xb, wa_ref[...], (((1,), (0,)), ((), ())),
                         precision=_PREC,
                         preferred_element_type=jnp.float32) + b_ref[...]
  wb = wb_ref[...]
  h = None
  for j in range(KNN):
    dj = xjt_ref[j] - xb
    hj = base + lax.dot_general(dj, wb, (((1,), (0,)), ((), ())),
                                precision=_PREC,
                                preferred_element_type=jnp.float32)
    hj = jnp.maximum(hj, 0.0)
    h = hj if h is None else jnp.maximum(h, hj)
  out_ref[...] = h


def _edge_mlp_final_body(xb_ref, xjt_ref, wa_ref, wb_ref, b_ref,
                         fw1_ref, fb1_ref, fw2_ref, fb2_ref, out_ref):
  xb = xb_ref[...]
  base = lax.dot_general(xb, wa_ref[...], (((1,), (0,)), ((), ())),
                         precision=_PREC,
                         preferred_element_type=jnp.float32) + b_ref[...]
  wb = wb_ref[...]
  h = None
  for j in range(KNN):
    dj = xjt_ref[j] - xb
    hj = base + lax.dot_general(dj, wb, (((1,), (0,)), ((), ())),
                                precision=_PREC,
                                preferred_element_type=jnp.float32)
    hj = jnp.maximum(hj, 0.0)
    h = hj if h is None else jnp.maximum(h, hj)
  t1 = lax.dot_general(h, fw1_ref[...], (((1,), (0,)), ((), ())),
                       precision=_PREC,
                       preferred_element_type=jnp.float32) + fb1_ref[...]
  t1 = jnp.maximum(t1, 0.0)
  out_ref[...] = lax.dot_general(t1, fw2_ref[...], (((1,), (0,)), ((), ())),
                                 precision=_PREC,
                                 preferred_element_type=jnp.float32) + fb2_ref[...]


def _edge_mlp(xp, xjt, wa, wb, b):
  dp = xp.shape[1]
  c = wa.shape[1]
  return pl.pallas_call(
      _edge_mlp_body,
      grid=(NB,),
      in_specs=[
          pl.BlockSpec((ROWS, dp), lambda i: (i, 0)),
          pl.BlockSpec((KNN, ROWS, dp), lambda i: (0, i, 0)),
          pl.BlockSpec((dp, c), lambda i: (0, 0)),
          pl.BlockSpec((dp, c), lambda i: (0, 0)),
          pl.BlockSpec((1, c), lambda i: (0, 0)),
      ],
      out_specs=pl.BlockSpec((ROWS, c), lambda i: (i, 0)),
      out_shape=jax.ShapeDtypeStruct((NP, c), jnp.float32),
  )(xp, xjt, wa, wb, b)


def _edge_mlp_final(xp, xjt, wa, wb, b, fw1, fb1, fw2, fb2):
  dp = xp.shape[1]
  c = wa.shape[1]
  co = fw2.shape[1]
  return pl.pallas_call(
      _edge_mlp_final_body,
      grid=(NB,),
      in_specs=[
          pl.BlockSpec((ROWS, dp), lambda i: (i, 0)),
          pl.BlockSpec((KNN, ROWS, dp), lambda i: (0, i, 0)),
          pl.BlockSpec((dp, c), lambda i: (0, 0)),
          pl.BlockSpec((dp, c), lambda i: (0, 0)),
          pl.BlockSpec((1, c), lambda i: (0, 0)),
          pl.BlockSpec((c, c), lambda i: (0, 0)),
          pl.BlockSpec((1, c), lambda i: (0, 0)),
          pl.BlockSpec((c, co), lambda i: (0, 0)),
          pl.BlockSpec((1, co), lambda i: (0, 0)),
      ],
      out_specs=pl.BlockSpec((ROWS, co), lambda i: (i, 0)),
      out_shape=jax.ShapeDtypeStruct((NP, co), jnp.float32),
  )(xp, xjt, wa, wb, b, fw1, fb1, fw2, fb2)


def kernel(x, edge_index, W1, b1, W2, b2, fW1, fb1, fW2, fb2):
  del edge_index

  dp1 = 16
  xp = jnp.zeros((NP, dp1), jnp.float32).at[:N, :3].set(x)
  idx1 = _topk(xp)
  idx1_flat = idx1.T.reshape(-1)
  xj1 = _sc_gather(xp, idx1_flat).reshape(KNN, NP, dp1)
  w1a = jnp.zeros((dp1, 64), jnp.float32).at[:3].set(W1[:3])
  w1b = jnp.zeros((dp1, 64), jnp.float32).at[:3].set(W1[3:])
  h1 = _edge_mlp(xp, xj1, w1a, w1b, b1.reshape(1, -1))

  idx2 = _topk(h1)
  idx2_flat = idx2.T.reshape(-1)
  xj2 = _sc_gather(h1, idx2_flat).reshape(KNN, NP, 64)
  out = _edge_mlp_final(h1, xj2, W2[:64], W2[64:], b2.reshape(1, -1),
                        fW1, fb1.reshape(1, -1), fW2, fb2.reshape(1, -1))
  return out[:N]


# speedup vs baseline: 7.8013x; 1.0365x over previous
import functools

import jax
import jax.numpy as jnp
from jax import lax
from jax.experimental import pallas as pl
from jax.experimental.pallas import tpu as pltpu
from jax.experimental.pallas import tpu_sc as plsc

N = 10000
NP = 10240
KNN = 20
ROWS = 512
NB = NP // ROWS

_PREC = jax.lax.Precision.DEFAULT


CW = 2560
NCH = NP // CW


def _extract(d, ids, nsel):
  w = d.shape[1]
  if w % 128 == 0 and w > 128:
    np_ = w // 128
    pd = [d[:, k * 128:(k + 1) * 128] for k in range(np_)]
    pc = [ids[:, k * 128:(k + 1) * 128] for k in range(np_)]
  else:
    pd, pc = [d], [ids]
  vals, idxs = [], []
  for _ in range(nsel):
    dm, cm = pd[0], pc[0]
    for k in range(1, len(pd)):
      keep = dm <= pd[k]
      cm = jnp.where(keep, cm, pc[k])
      dm = jnp.minimum(dm, pd[k])
    m = jnp.min(dm, axis=1, keepdims=True)
    am = jnp.min(jnp.where(dm <= m, cm, jnp.float32(3e38)),
                 axis=1, keepdims=True)
    vals.append(m)
    idxs.append(am)
    pd = [jnp.where(pc[k] == am, jnp.inf, pd[k]) for k in range(len(pd))]
  return jnp.concatenate(vals, axis=1), jnp.concatenate(idxs, axis=1)


def _topk_chunk_body(xb_ref, xtc_ref, cv_ref, ci_ref):
  i = pl.program_id(0)
  c = pl.program_id(1)
  xb = xb_ref[...]
  xtc = xtc_ref[...]
  sq_i = jnp.sum(xb * xb, axis=1, keepdims=True)
  sq_j = jnp.sum(xtc * xtc, axis=0, keepdims=True)
  cross = lax.dot_general(xb, xtc, (((1,), (0,)), ((), ())),
                          precision=_PREC,
                          preferred_element_type=jnp.float32)
  d = sq_i + sq_j - 2.0 * cross
  col = c * CW + lax.broadcasted_iota(jnp.int32, (ROWS, CW), 1)
  row = i * ROWS + lax.broadcasted_iota(jnp.int32, (ROWS, CW), 0)
  d = jnp.where(jnp.logical_or(col == row, col >= N), jnp.inf, d)
  colf = col.astype(jnp.float32)
  cvals, cidx = _extract(d, colf, KNN)
  cv_ref[0] = cvals
  ci_ref[0] = cidx


def _topk_merge_body(cv_ref, ci_ref, idx_ref):
  cvs = cv_ref[...]
  cis = ci_ref[...]
  v = jnp.concatenate([cvs[k] for k in range(NCH)], axis=1)
  ids = jnp.concatenate([cis[k] for k in range(NCH)], axis=1)
  _, midx = _extract(v, ids, KNN)
  idx_ref[...] = midx.astype(jnp.int32)


def _topk(xp):
  dp = xp.shape[1]
  xt = xp.T
  cv, ci = pl.pallas_call(
      _topk_chunk_body,
      grid=(NB, NCH),
      in_specs=[
          pl.BlockSpec((ROWS, dp), lambda i, c: (i, 0)),
          pl.BlockSpec((dp, CW), lambda i, c: (0, c)),
      ],
      out_specs=[
          pl.BlockSpec((1, ROWS, KNN), lambda i, c: (c, i, 0)),
          pl.BlockSpec((1, ROWS, KNN), lambda i, c: (c, i, 0)),
      ],
      out_shape=[
          jax.ShapeDtypeStruct((NCH, NP, KNN), jnp.float32),
          jax.ShapeDtypeStruct((NCH, NP, KNN), jnp.float32),
      ],
  )(xp, xt)
  return pl.pallas_call(
      _topk_merge_body,
      grid=(NB,),
      in_specs=[
          pl.BlockSpec((NCH, ROWS, KNN), lambda i: (0, i, 0)),
          pl.BlockSpec((NCH, ROWS, KNN), lambda i: (0, i, 0)),
      ],
      out_specs=pl.BlockSpec((ROWS, KNN), lambda i: (i, 0)),
      out_shape=jax.ShapeDtypeStruct((NP, KNN), jnp.int32),
  )(cv, ci)


def _sc_gather(table, idx_flat):
  B = idx_flat.shape[0]
  dp = table.shape[1]
  NW = 32
  b_per_w = B // NW
  ch = 640
  n_ch = b_per_w // ch
  mesh = plsc.VectorSubcoreMesh(core_axis_name="c", subcore_axis_name="s")

  @functools.partial(
      pl.kernel, mesh=mesh,
      compiler_params=pltpu.CompilerParams(use_tc_tiling_on_sc=False),
      out_type=jax.ShapeDtypeStruct((B, dp), jnp.float32),
      scratch_types=[
          pltpu.VMEM((ch,), jnp.int32),
          pltpu.VMEM((ch, dp), jnp.float32),
          pltpu.SemaphoreType.DMA,
      ],
  )
  def k(table_hbm, idx_hbm, out_hbm, idx_v, rows_v, sem):
    wid = lax.axis_index("s") * 2 + lax.axis_index("c")

    def body(t, carry):
      base = wid * b_per_w + t * ch
      pltpu.sync_copy(idx_hbm.at[pl.ds(base, ch)], idx_v)
      pltpu.async_copy(table_hbm.at[idx_v], rows_v, sem).wait()
      pltpu.sync_copy(rows_v, out_hbm.at[pl.ds(base, ch)])
      return carry

    lax.fori_loop(0, n_ch, body, 0)

  return k(table, idx_flat)


def _edge_mlp_body(xb_ref, xjt_ref, wa_ref, wb_ref, b_ref, out_ref):
  xb = xb_ref[...]
  base = lax.dot_general(xb, wa_ref[...], (((1,), (0,)), ((), ())),
                         precision=_PREC,
                         preferred_element_type=jnp.float32) + b_ref[...]
  wb = wb_ref[...]
  h = None
  for j in range(KNN):
    dj = xjt_ref[j] - xb
    hj = base + lax.dot_general(dj, wb, (((1,), (0,)), ((), ())),
                                precision=_PREC,
                                preferred_element_type=jnp.float32)
    hj = jnp.maximum(hj, 0.0)
    h = hj if h is None else jnp.maximum(h, hj)
  out_ref[...] = h


def _edge_mlp_final_body(xb_ref, xjt_ref, wa_ref, wb_ref, b_ref,
                         fw1_ref, fb1_ref, fw2_ref, fb2_ref, out_ref):
  xb = xb_ref[...]
  base = lax.dot_general(xb, wa_ref[...], (((1,), (0,)), ((), ())),
                         precision=_PREC,
                         preferred_element_type=jnp.float32) + b_ref[...]
  wb = wb_ref[...]
  h = None
  for j in range(KNN):
    dj = xjt_ref[j] - xb
    hj = base + lax.dot_general(dj, wb, (((1,), (0,)), ((), ())),
                                precision=_PREC,
                                preferred_element_type=jnp.float32)
    hj = jnp.maximum(hj, 0.0)
    h = hj if h is None else jnp.maximum(h, hj)
  t1 = lax.dot_general(h, fw1_ref[...], (((1,), (0,)), ((), ())),
                       precision=_PREC,
                       preferred_element_type=jnp.float32) + fb1_ref[...]
  t1 = jnp.maximum(t1, 0.0)
  out_ref[...] = lax.dot_general(t1, fw2_ref[...], (((1,), (0,)), ((), ())),
                                 precision=_PREC,
                                 preferred_element_type=jnp.float32) + fb2_ref[...]


def _edge_mlp(xp, xjt, wa, wb, b):
  dp = xp.shape[1]
  c = wa.shape[1]
  return pl.pallas_call(
      _edge_mlp_body,
      grid=(NB,),
      in_specs=[
          pl.BlockSpec((ROWS, dp), lambda i: (i, 0)),
          pl.BlockSpec((KNN, ROWS, dp), lambda i: (0, i, 0)),
          pl.BlockSpec((dp, c), lambda i: (0, 0)),
          pl.BlockSpec((dp, c), lambda i: (0, 0)),
          pl.BlockSpec((1, c), lambda i: (0, 0)),
      ],
      out_specs=pl.BlockSpec((ROWS, c), lambda i: (i, 0)),
      out_shape=jax.ShapeDtypeStruct((NP, c), jnp.float32),
  )(xp, xjt, wa, wb, b)


def _edge_mlp_final(xp, xjt, wa, wb, b, fw1, fb1, fw2, fb2):
  dp = xp.shape[1]
  c = wa.shape[1]
  co = fw2.shape[1]
  return pl.pallas_call(
      _edge_mlp_final_body,
      grid=(NB,),
      in_specs=[
          pl.BlockSpec((ROWS, dp), lambda i: (i, 0)),
          pl.BlockSpec((KNN, ROWS, dp), lambda i: (0, i, 0)),
          pl.BlockSpec((dp, c), lambda i: (0, 0)),
          pl.BlockSpec((dp, c), lambda i: (0, 0)),
          pl.BlockSpec((1, c), lambda i: (0, 0)),
          pl.BlockSpec((c, c), lambda i: (0, 0)),
          pl.BlockSpec((1, c), lambda i: (0, 0)),
          pl.BlockSpec((c, co), lambda i: (0, 0)),
          pl.BlockSpec((1, co), lambda i: (0, 0)),
      ],
      out_specs=pl.BlockSpec((ROWS, co), lambda i: (i, 0)),
      out_shape=jax.ShapeDtypeStruct((NP, co), jnp.float32),
  )(xp, xjt, wa, wb, b, fw1, fb1, fw2, fb2)


def kernel(x, edge_index, W1, b1, W2, b2, fW1, fb1, fW2, fb2):
  del edge_index

  dp1 = 16
  xp = jnp.zeros((NP, dp1), jnp.float32).at[:N, :3].set(x)
  idx1 = _topk(xp)
  idx1_flat = idx1.T.reshape(-1)
  xj1 = _sc_gather(xp, idx1_flat).reshape(KNN, NP, dp1)
  w1a = jnp.zeros((dp1, 64), jnp.float32).at[:3].set(W1[:3])
  w1b = jnp.zeros((dp1, 64), jnp.float32).at[:3].set(W1[3:])
  h1 = _edge_mlp(xp, xj1, w1a, w1b, b1.reshape(1, -1))

  idx2 = _topk(h1)
  idx2_flat = idx2.T.reshape(-1)
  xj2 = _sc_gather(h1, idx2_flat).reshape(KNN, NP, 64)
  out = _edge_mlp_final(h1, xj2, W2[:64], W2[64:], b2.reshape(1, -1),
                        fW1, fb1.reshape(1, -1), fW2, fb2.reshape(1, -1))
  return out[:N]
